# baseline (speedup 1.0000x reference)
import functools

import jax
import jax.numpy as jnp
from jax import lax
from jax.experimental import pallas as pl
from jax.experimental.pallas import tpu as pltpu
from jax.experimental.pallas import tpu_sc as plsc

_NBUF = 4
_LANES = 16


def kernel(x, table):
    B, S = x.shape
    V, D = table.shape

    info = plsc.get_sparse_core_info()
    NC, NS = info.num_cores, info.num_subcores
    NW = NC * NS
    W = B // NW

    steady = S - 2 * _NBUF
    assert W == 128 and D % 8 == 0
    assert steady % _NBUF == 0 and steady >= 0

    x_t = x.astype(jnp.int32).T

    mesh = plsc.VectorSubcoreMesh(core_axis_name="c", subcore_axis_name="s")

    @functools.partial(
        pl.kernel,
        mesh=mesh,
        compiler_params=pltpu.CompilerParams(
            use_tc_tiling_on_sc=False, needs_layout_passes=False
        ),
        out_type=jax.ShapeDtypeStruct((S, D // 8, NW, 8, W), jnp.float32),
        scratch_types=[
            pltpu.VMEM((S, W), jnp.int32),
            pltpu.VMEM((_NBUF, W, D), jnp.float32),
            pltpu.VMEM((_NBUF, D // 8, 8, W), jnp.float32),
            [pltpu.SemaphoreType.DMA] * _NBUF,
            [pltpu.SemaphoreType.DMA] * _NBUF,
        ],
    )
    def emb(idx_hbm, tab_hbm, out_hbm, idx_all, gbuf, obuf, gsems, osems):
        wid = lax.axis_index("s") * NC + lax.axis_index("c")
        b0 = wid * W

        pltpu.sync_copy(idx_hbm.at[:, pl.ds(b0, W)], idx_all)

        lane = lax.iota(jnp.int32, _LANES)

        def fire(g, b):
            pltpu.async_copy(tab_hbm.at[idx_all.at[g]], gbuf.at[b], gsems[b])

        def wait_g(b):
            pltpu.make_async_copy(tab_hbm.at[pl.ds(0, W)], gbuf.at[b], gsems[b]).wait()

        def transpose(b):
            def per_group(k, carry):
                rows = k * _LANES + lane

                def per_d(d, c2):
                    vals = plsc.load_gather(
                        gbuf.at[b], [rows, jnp.broadcast_to(d, (_LANES,))]
                    )
                    obuf[b, d >> 3, d & 7, pl.ds(k * _LANES, _LANES)] = vals
                    return c2

                lax.fori_loop(0, D, per_d, 0)
                return carry

            lax.fori_loop(0, W // _LANES, per_group, 0)

        def fire_w(g, b):
            pltpu.async_copy(obuf.at[b], out_hbm.at[g, :, wid], osems[b])

        def wait_w(b):
            pltpu.make_async_copy(obuf.at[b], out_hbm.at[0, :, wid], osems[b]).wait()

        for b in range(_NBUF):
            fire(b, b)
        for b in range(_NBUF):
            wait_g(b)
            transpose(b)
            fire_w(b, b)
            fire(b + _NBUF, b)

        def body(i, carry):
            gg = _NBUF + i * _NBUF
            for b in range(_NBUF):
                g = gg + b
                wait_g(b)
                wait_w(b)
                transpose(b)
                fire_w(g, b)
                fire(g + _NBUF, b)
            return carry

        lax.fori_loop(0, steady // _NBUF, body, 0)

        for b in range(_NBUF):
            g = S - _NBUF + b
            wait_g(b)
            wait_w(b)
            transpose(b)
            fire_w(g, b)
        for b in range(_NBUF):
            wait_w(b)

    out5 = emb(x_t, table)
    return out5.transpose(2, 4, 0, 1, 3).reshape(B, S, D)


# speedup vs baseline: 1.0048x; 1.0048x over previous
import functools

import jax
import jax.numpy as jnp
from jax import lax
from jax.experimental import pallas as pl
from jax.experimental.pallas import tpu as pltpu
from jax.experimental.pallas import tpu_sc as plsc

_NBUF = 4
_LANES = 16


def kernel(x, table):
    B, S = x.shape
    V, D = table.shape

    info = plsc.get_sparse_core_info()
    NC, NS = info.num_cores, info.num_subcores
    NW = NC * NS
    W = B // NW

    steady = S - 2 * _NBUF
    assert W == 128 and D % 8 == 0
    assert steady % _NBUF == 0 and steady >= 0

    x_t = x.astype(jnp.int32).T

    mesh = plsc.VectorSubcoreMesh(core_axis_name="c", subcore_axis_name="s")

    @functools.partial(
        pl.kernel,
        mesh=mesh,
        compiler_params=pltpu.CompilerParams(
            use_tc_tiling_on_sc=False, needs_layout_passes=False
        ),
        out_type=jax.ShapeDtypeStruct((S, D // 8, NW, 8, W), jnp.float32),
        scratch_types=[
            pltpu.VMEM((S, W), jnp.int32),
            pltpu.VMEM((_NBUF, W, D), jnp.float32),
            pltpu.VMEM((_NBUF, D // 8, 8, W), jnp.float32),
            [pltpu.SemaphoreType.DMA] * _NBUF,
            [pltpu.SemaphoreType.DMA] * _NBUF,
        ],
    )
    def emb(idx_hbm, tab_hbm, out_hbm, idx_all, gbuf, obuf, gsems, osems):
        wid = lax.axis_index("s") * NC + lax.axis_index("c")
        b0 = wid * W

        pltpu.sync_copy(idx_hbm.at[:, pl.ds(b0, W)], idx_all)

        lane = lax.iota(jnp.int32, _LANES)

        def fire(g, b):
            pltpu.async_copy(tab_hbm.at[idx_all.at[g]], gbuf.at[b], gsems[b])

        def wait_g(b):
            pltpu.make_async_copy(tab_hbm.at[pl.ds(0, W)], gbuf.at[b], gsems[b]).wait()

        rows_k = [k * _LANES + lane for k in range(W // _LANES)]

        def transpose(b):
            def per_d(d, carry):
                cols = jnp.broadcast_to(d, (_LANES,))
                dt = d >> 3
                dr = d & 7
                for k in range(W // _LANES):
                    vals = plsc.load_gather(gbuf.at[b], [rows_k[k], cols])
                    obuf[b, dt, dr, pl.ds(k * _LANES, _LANES)] = vals
                return carry

            lax.fori_loop(0, D, per_d, 0)

        def fire_w(g, b):
            pltpu.async_copy(obuf.at[b], out_hbm.at[g, :, wid], osems[b])

        def wait_w(b):
            pltpu.make_async_copy(obuf.at[b], out_hbm.at[0, :, wid], osems[b]).wait()

        for b in range(_NBUF):
            fire(b, b)
        for b in range(_NBUF):
            wait_g(b)
            transpose(b)
            fire_w(b, b)
            fire(b + _NBUF, b)

        def body(i, carry):
            gg = _NBUF + i * _NBUF
            for b in range(_NBUF):
                g = gg + b
                wait_g(b)
                wait_w(b)
                transpose(b)
                fire_w(g, b)
                fire(g + _NBUF, b)
            return carry

        lax.fori_loop(0, steady // _NBUF, body, 0)

        for b in range(_NBUF):
            g = S - _NBUF + b
            wait_g(b)
            wait_w(b)
            transpose(b)
            fire_w(g, b)
        for b in range(_NBUF):
            wait_w(b)

    out5 = emb(x_t, table)
    return out5.transpose(2, 4, 0, 1, 3).reshape(B, S, D)


# speedup vs baseline: 2.6903x; 2.6775x over previous
import functools

import jax
import jax.numpy as jnp
from jax import lax
from jax.experimental import pallas as pl
from jax.experimental.pallas import tpu as pltpu
from jax.experimental.pallas import tpu_sc as plsc

_NBUF = 4
_LANES = 16


def kernel(x, table):
    B, S = x.shape
    V, D = table.shape

    info = plsc.get_sparse_core_info()
    NC, NS = info.num_cores, info.num_subcores
    NW = NC * NS
    W = B // NW

    steady = S - 2 * _NBUF
    assert W == 128 and D % 8 == 0
    assert steady % _NBUF == 0 and steady >= 0

    x_t = x.astype(jnp.int32).T

    mesh = plsc.VectorSubcoreMesh(core_axis_name="c", subcore_axis_name="s")

    @functools.partial(
        pl.kernel,
        mesh=mesh,
        compiler_params=pltpu.CompilerParams(
            use_tc_tiling_on_sc=False, needs_layout_passes=False
        ),
        out_type=jax.ShapeDtypeStruct((S, D // 8, NW, 8, W), jnp.float32),
        scratch_types=[
            pltpu.VMEM((S, W), jnp.int32),
            pltpu.VMEM((_NBUF, W, D), jnp.float32),
            pltpu.VMEM((_NBUF, D // 8, 8, W + 1), jnp.float32),
            [pltpu.SemaphoreType.DMA] * _NBUF,
            [pltpu.SemaphoreType.DMA] * _NBUF,
        ],
    )
    def emb(idx_hbm, tab_hbm, out_hbm, idx_all, gbuf, obuf, gsems, osems):
        wid = lax.axis_index("s") * NC + lax.axis_index("c")
        b0 = wid * W

        pltpu.sync_copy(idx_hbm.at[:, pl.ds(b0, W)], idx_all)

        lane = lax.iota(jnp.int32, _LANES)

        def fire(g, b):
            pltpu.async_copy(tab_hbm.at[idx_all.at[g]], gbuf.at[b], gsems[b])

        def wait_g(b):
            pltpu.make_async_copy(tab_hbm.at[pl.ds(0, W)], gbuf.at[b], gsems[b]).wait()

        dq = [q * _LANES + lane for q in range(D // _LANES)]
        dt_q = [lax.shift_right_logical(d, 3) for d in dq]
        dr_q = [lax.bitwise_and(d, 7) for d in dq]

        def transpose(b):
            def per_row(r, carry):
                rb = jnp.broadcast_to(r, (_LANES,))
                for q in range(D // _LANES):
                    vals = gbuf[b, r, pl.ds(q * _LANES, _LANES)]
                    plsc.store_scatter(obuf.at[b], [dt_q[q], dr_q[q], rb], vals)
                return carry

            lax.fori_loop(0, W, per_row, 0)

        def fire_w(g, b):
            pltpu.async_copy(
                obuf.at[b, :, :, pl.ds(0, W)], out_hbm.at[g, :, wid], osems[b]
            )

        def wait_w(b):
            pltpu.make_async_copy(
                obuf.at[b, :, :, pl.ds(0, W)], out_hbm.at[0, :, wid], osems[b]
            ).wait()

        for b in range(_NBUF):
            fire(b, b)
        for b in range(_NBUF):
            wait_g(b)
            transpose(b)
            fire_w(b, b)
            fire(b + _NBUF, b)

        def body(i, carry):
            gg = _NBUF + i * _NBUF
            for b in range(_NBUF):
                g = gg + b
                wait_g(b)
                wait_w(b)
                transpose(b)
                fire_w(g, b)
                fire(g + _NBUF, b)
            return carry

        lax.fori_loop(0, steady // _NBUF, body, 0)

        for b in range(_NBUF):
            g = S - _NBUF + b
            wait_g(b)
            wait_w(b)
            transpose(b)
            fire_w(g, b)
        for b in range(_NBUF):
            wait_w(b)

    out5 = emb(x_t, table)
    return out5.transpose(2, 4, 0, 1, 3).reshape(B, S, D)


# speedup vs baseline: 2.8329x; 1.0530x over previous
import functools

import jax
import jax.numpy as jnp
from jax import lax
from jax.experimental import pallas as pl
from jax.experimental.pallas import tpu as pltpu
from jax.experimental.pallas import tpu_sc as plsc

_NBUF = 4
_LANES = 16
_UNROLL = 4


def kernel(x, table):
    B, S = x.shape
    V, D = table.shape

    info = plsc.get_sparse_core_info()
    NC, NS = info.num_cores, info.num_subcores
    NW = NC * NS
    W = B // NW
    ST = S // 8

    steady = S - 2 * _NBUF
    assert W == 128 and D % _LANES == 0 and S % 8 == 0
    assert steady % _NBUF == 0 and steady >= 0 and W % _UNROLL == 0

    x4 = x.astype(jnp.int32).reshape(NW, W, ST, 8).transpose(2, 0, 3, 1)

    mesh = plsc.VectorSubcoreMesh(core_axis_name="c", subcore_axis_name="s")

    @functools.partial(
        pl.kernel,
        mesh=mesh,
        compiler_params=pltpu.CompilerParams(
            use_tc_tiling_on_sc=False, needs_layout_passes=False
        ),
        out_type=jax.ShapeDtypeStruct((S, D // 8, NW, 8, W), jnp.float32),
        scratch_types=[
            pltpu.VMEM((ST, 8, W), jnp.int32),
            pltpu.VMEM((_NBUF, W, D), jnp.float32),
            pltpu.VMEM((_NBUF, D // 8, 8, W + 1), jnp.float32),
            [pltpu.SemaphoreType.DMA] * _NBUF,
            [pltpu.SemaphoreType.DMA] * _NBUF,
        ],
    )
    def emb(idx_hbm, tab_hbm, out_hbm, idx_all, gbuf, obuf, gsems, osems):
        wid = lax.axis_index("s") * NC + lax.axis_index("c")

        pltpu.sync_copy(idx_hbm.at[:, wid], idx_all)

        lane = lax.iota(jnp.int32, _LANES)

        def fire(g, b):
            pltpu.async_copy(
                tab_hbm.at[idx_all.at[g >> 3, g & 7]], gbuf.at[b], gsems[b]
            )

        def wait_g(b):
            pltpu.make_async_copy(tab_hbm.at[pl.ds(0, W)], gbuf.at[b], gsems[b]).wait()

        dq = [q * _LANES + lane for q in range(D // _LANES)]
        dt_q = [lax.shift_right_logical(d, 3) for d in dq]
        dr_q = [lax.bitwise_and(d, 7) for d in dq]
        one = jnp.broadcast_to(jnp.int32(1), (_LANES,))

        def transpose(b):
            def per_row(i, rb):
                r0 = i * _UNROLL
                for j in range(_UNROLL):
                    for q in range(D // _LANES):
                        vals = gbuf[b, r0 + j, pl.ds(q * _LANES, _LANES)]
                        plsc.store_scatter(obuf.at[b], [dt_q[q], dr_q[q], rb], vals)
                    rb = rb + one
                return rb

            lax.fori_loop(0, W // _UNROLL, per_row, jnp.broadcast_to(0, (_LANES,)))

        def fire_w(g, b):
            pltpu.async_copy(
                obuf.at[b, :, :, pl.ds(0, W)], out_hbm.at[g, :, wid], osems[b]
            )

        def wait_w(b):
            pltpu.make_async_copy(
                obuf.at[b, :, :, pl.ds(0, W)], out_hbm.at[0, :, wid], osems[b]
            ).wait()

        for b in range(_NBUF):
            fire(b, b)
        for b in range(_NBUF):
            wait_g(b)
            transpose(b)
            fire_w(b, b)
            fire(b + _NBUF, b)

        def body(i, carry):
            gg = _NBUF + i * _NBUF
            for b in range(_NBUF):
                g = gg + b
                wait_g(b)
                wait_w(b)
                transpose(b)
                fire_w(g, b)
                fire(g + _NBUF, b)
            return carry

        lax.fori_loop(0, steady // _NBUF, body, 0)

        for b in range(_NBUF):
            g = S - _NBUF + b
            wait_g(b)
            wait_w(b)
            transpose(b)
            fire_w(g, b)
        for b in range(_NBUF):
            wait_w(b)

    out5 = emb(x4, table)
    return out5.transpose(2, 4, 0, 1, 3).reshape(B, S, D)
